# Initial kernel scaffold; baseline (speedup 1.0000x reference)
import jax
import jax.numpy as jnp
from jax.experimental import pallas as pl


def kernel(all_feat):
    raise NotImplementedError("write your pallas kernel here")


# speedup vs baseline: 4.9684x; 4.9684x over previous
import functools

import jax
import jax.numpy as jnp
from jax import lax
from jax.experimental import pallas as pl
from jax.experimental.pallas import tpu as pltpu
from jax.experimental.pallas import tpu_sc as plsc

N = 16384
D = 16
K = 5
ROWS = 256
CHUNK = 2048
NEG = -2.0

NUM_WORKERS = 32
ROWS_SC = N // NUM_WORKERS
TERMS = ROWS_SC * K
GBLK = 128
NBLK = TERMS // GBLK


def _norm_body(x_ref, o_ref):
    x = x_ref[...]
    n = jnp.sqrt(jnp.sum(x * x, axis=1, keepdims=True))
    o_ref[...] = x / jnp.maximum(n, 1e-12)


def _topk_body(rows_ref, colsT_ref, w_ref, idx_ref):
    rows = rows_ref[...]
    run_v = jnp.full((ROWS, K), NEG, jnp.float32)
    run_i = jnp.zeros((ROWS, K), jnp.int32)
    for c in range(N // CHUNK):
        base = c * CHUNK
        cols = colsT_ref[:, pl.ds(base, CHUNK)]
        sim = jnp.dot(rows, cols, preferred_element_type=jnp.float32)
        gcol = base + lax.broadcasted_iota(jnp.int32, (ROWS, CHUNK), 1)
        vals = sim
        vs, ix = [], []
        for _ in range(K):
            m = jnp.max(vals, axis=1, keepdims=True)
            eq = vals == m
            mi = jnp.min(jnp.where(eq, gcol, N), axis=1, keepdims=True)
            vals = jnp.where(eq, NEG, vals)
            vs.append(m)
            ix.append(mi)
        cand_v = jnp.concatenate([run_v] + vs, axis=1)
        cand_i = jnp.concatenate([run_i] + ix, axis=1)
        vs, ix = [], []
        for _ in range(K):
            m = jnp.max(cand_v, axis=1, keepdims=True)
            eq = cand_v == m
            mi = jnp.min(jnp.where(eq, cand_i, N), axis=1, keepdims=True)
            cand_v = jnp.where(eq & (cand_i == mi), NEG, cand_v)
            vs.append(m)
            ix.append(mi)
        run_v = jnp.concatenate(vs, axis=1)
        run_i = jnp.concatenate(ix, axis=1)
    w = run_v * run_v * jnp.float32(0.2)
    w_ref[...] = jnp.concatenate([w, jnp.zeros((ROWS, 8 - K), jnp.float32)], axis=1)
    idx_ref[...] = run_i


def _topk(norm_feat, norm_feat_T):
    return pl.pallas_call(
        _topk_body,
        grid=(N // ROWS,),
        in_specs=[
            pl.BlockSpec((ROWS, D), lambda i: (i, 0)),
            pl.BlockSpec((D, N), lambda i: (0, 0)),
        ],
        out_specs=[
            pl.BlockSpec((ROWS, 8), lambda i: (i, 0)),
            pl.BlockSpec((ROWS, K), lambda i: (i, 0)),
        ],
        out_shape=[
            jax.ShapeDtypeStruct((N, 8), jnp.float32),
            jax.ShapeDtypeStruct((N, K), jnp.int32),
        ],
    )(norm_feat, norm_feat_T)


def _combine_body(table_hbm, idx_hbm, w_hbm, out_hbm, idx_v, rows_v, w_v, out_v, sem):
    c = lax.axis_index("c")
    s = lax.axis_index("s")
    wid = s * 2 + c
    pltpu.sync_copy(idx_hbm.at[wid], idx_v)
    pltpu.sync_copy(w_hbm.at[wid], w_v)
    copies = []
    for b in range(NBLK):
        copies.append(
            pltpu.async_copy(
                table_hbm.at[idx_v.at[b]],
                rows_v.at[pl.ds(b * GBLK, GBLK)],
                sem,
            )
        )
    for cp in copies:
        cp.wait()

    def group_body(g, _):
        wvecs = [w_v[pl.ds(g * 128 + k * 16, 16)] for k in range(8)]
        for rr in range(16):
            acc = jnp.zeros((D,), jnp.float32)
            for j in range(K):
                t = g * (16 * K) + rr * K + j
                p = rr * 8 + j
                acc = acc + rows_v[t, :] * wvecs[p // 16][p % 16]
            out_v[g * 16 + rr, :] = acc
        return 0

    lax.fori_loop(0, ROWS_SC // 16, group_body, 0)
    pltpu.sync_copy(out_v, out_hbm.at[pl.ds(wid * ROWS_SC, ROWS_SC)])


@functools.lru_cache(maxsize=1)
def _combine_kernel():
    return pl.kernel(
        _combine_body,
        out_type=jax.ShapeDtypeStruct((N, D), jnp.float32),
        mesh=plsc.VectorSubcoreMesh(core_axis_name="c", subcore_axis_name="s"),
        scratch_types=[
            pltpu.VMEM((NBLK, GBLK), jnp.int32),
            pltpu.VMEM((TERMS, D), jnp.float32),
            pltpu.VMEM((ROWS_SC * 8,), jnp.float32),
            pltpu.VMEM((ROWS_SC, D), jnp.float32),
            pltpu.SemaphoreType.DMA,
        ],
        compiler_params=pltpu.CompilerParams(use_tc_tiling_on_sc=False),
    )


def kernel(all_feat):
    norm_feat = pl.pallas_call(
        _norm_body,
        out_shape=jax.ShapeDtypeStruct((N, D), jnp.float32),
    )(all_feat)
    w, idx = _topk(norm_feat, norm_feat.T)
    idx_r = idx.reshape(NUM_WORKERS, NBLK, GBLK)
    w_r = w.reshape(NUM_WORKERS, ROWS_SC * 8)
    return _combine_kernel()(all_feat, idx_r, w_r)


# speedup vs baseline: 7.0848x; 1.4260x over previous
import functools

import jax
import jax.numpy as jnp
from jax import lax
from jax.experimental import pallas as pl
from jax.experimental.pallas import tpu as pltpu
from jax.experimental.pallas import tpu_sc as plsc

N = 16384
D = 16
K = 5
ROWS = 256
CHUNK = 2048
NEG = -2.0

NUM_WORKERS = 32
ROWS_SC = N // NUM_WORKERS
TERMS = ROWS_SC * K
GBLK = 128
NBLK = TERMS // GBLK


def _norm_body(x_ref, o_ref):
    x = x_ref[...]
    n = jnp.sqrt(jnp.sum(x * x, axis=1, keepdims=True))
    o_ref[...] = x / jnp.maximum(n, 1e-12)


def _topk_body(rows_ref, colsT_ref, w_ref, idx_ref):
    pid = pl.program_id(0)
    rows = rows_ref[...]
    rowid = pid * ROWS + lax.broadcasted_iota(jnp.int32, (ROWS, 1), 0)
    self_v = jnp.sum(rows * rows, axis=1, keepdims=True)
    lidx = lax.broadcasted_iota(jnp.int32, (ROWS, CHUNK), 1)
    KO = K - 1
    vs, ix = [], []
    for c in range(N // CHUNK):
        base = c * CHUNK
        cols = colsT_ref[:, pl.ds(base, CHUNK)]
        sim = jnp.dot(rows, cols, preferred_element_type=jnp.float32)
        vals = jnp.where(lidx == rowid - base, NEG, sim)
        for r in range(KO):
            m = jnp.max(vals, axis=1, keepdims=True)
            eq = vals == m
            mi = jnp.min(jnp.where(eq, lidx, N), axis=1, keepdims=True)
            if r < KO - 1:
                vals = jnp.where(eq, NEG, vals)
            vs.append(m)
            ix.append(mi + base)
    cand_v = jnp.concatenate(vs, axis=1)
    cand_i = jnp.concatenate(ix, axis=1)
    vs, ix = [], []
    for _ in range(KO):
        m = jnp.max(cand_v, axis=1, keepdims=True)
        eq = cand_v == m
        mi = jnp.min(jnp.where(eq, cand_i, N), axis=1, keepdims=True)
        cand_v = jnp.where(eq & (cand_i == mi), NEG, cand_v)
        vs.append(m)
        ix.append(mi)
    top_v = jnp.concatenate([self_v] + vs, axis=1)
    top_i = jnp.concatenate([rowid] + ix, axis=1)
    top_i = jnp.clip(top_i, 0, N - 1)
    w = top_v * top_v * jnp.float32(0.2)
    w_ref[...] = jnp.concatenate([w, jnp.zeros((ROWS, 8 - K), jnp.float32)], axis=1)
    idx_ref[...] = top_i


def _topk(norm_feat, norm_feat_T):
    return pl.pallas_call(
        _topk_body,
        grid=(N // ROWS,),
        in_specs=[
            pl.BlockSpec((ROWS, D), lambda i: (i, 0)),
            pl.BlockSpec((D, N), lambda i: (0, 0)),
        ],
        out_specs=[
            pl.BlockSpec((ROWS, 8), lambda i: (i, 0)),
            pl.BlockSpec((ROWS, K), lambda i: (i, 0)),
        ],
        out_shape=[
            jax.ShapeDtypeStruct((N, 8), jnp.float32),
            jax.ShapeDtypeStruct((N, K), jnp.int32),
        ],
    )(norm_feat, norm_feat_T)


def _combine_body(table_hbm, idx_hbm, w_hbm, out_hbm, idx_v, rows_v, w_v, out_v, sem):
    c = lax.axis_index("c")
    s = lax.axis_index("s")
    wid = s * 2 + c
    pltpu.sync_copy(idx_hbm.at[wid], idx_v)
    pltpu.sync_copy(w_hbm.at[wid], w_v)
    copies = []
    for b in range(NBLK):
        copies.append(
            pltpu.async_copy(
                table_hbm.at[idx_v.at[b]],
                rows_v.at[pl.ds(b * GBLK, GBLK)],
                sem,
            )
        )
    for cp in copies:
        cp.wait()

    def group_body(g, _):
        wvecs = [w_v[pl.ds(g * 128 + k * 16, 16)] for k in range(8)]
        for rr in range(16):
            acc = jnp.zeros((D,), jnp.float32)
            for j in range(K):
                t = g * (16 * K) + rr * K + j
                p = rr * 8 + j
                acc = acc + rows_v[t, :] * wvecs[p // 16][p % 16]
            out_v[g * 16 + rr, :] = acc
        return 0

    lax.fori_loop(0, ROWS_SC // 16, group_body, 0)
    pltpu.sync_copy(out_v, out_hbm.at[pl.ds(wid * ROWS_SC, ROWS_SC)])


@functools.lru_cache(maxsize=1)
def _combine_kernel():
    return pl.kernel(
        _combine_body,
        out_type=jax.ShapeDtypeStruct((N, D), jnp.float32),
        mesh=plsc.VectorSubcoreMesh(core_axis_name="c", subcore_axis_name="s"),
        scratch_types=[
            pltpu.VMEM((NBLK, GBLK), jnp.int32),
            pltpu.VMEM((TERMS, D), jnp.float32),
            pltpu.VMEM((ROWS_SC * 8,), jnp.float32),
            pltpu.VMEM((ROWS_SC, D), jnp.float32),
            pltpu.SemaphoreType.DMA,
        ],
        compiler_params=pltpu.CompilerParams(use_tc_tiling_on_sc=False),
    )


def kernel(all_feat):
    norm_feat = pl.pallas_call(
        _norm_body,
        out_shape=jax.ShapeDtypeStruct((N, D), jnp.float32),
    )(all_feat)
    w, idx = _topk(norm_feat, norm_feat.T)
    idx_r = idx.reshape(NUM_WORKERS, NBLK, GBLK)
    w_r = w.reshape(NUM_WORKERS, ROWS_SC * 8)
    return _combine_kernel()(all_feat, idx_r, w_r)


# speedup vs baseline: 7.1640x; 1.0112x over previous
import functools

import jax
import jax.numpy as jnp
from jax import lax
from jax.experimental import pallas as pl
from jax.experimental.pallas import tpu as pltpu
from jax.experimental.pallas import tpu_sc as plsc

N = 16384
D = 16
K = 5
ROWS = 256
CHUNK = 2048
NEG = -2.0

NUM_WORKERS = 32
ROWS_SC = N // NUM_WORKERS
TERMS = ROWS_SC * K
GBLK = 128
NBLK = TERMS // GBLK


def _norm_body(x_ref, o_ref):
    x = x_ref[...]
    n = jnp.sqrt(jnp.sum(x * x, axis=1, keepdims=True))
    o_ref[...] = x / jnp.maximum(n, 1e-12)


def _topk_body(rows_ref, colsT_ref, w_ref, idx_ref):
    pid = pl.program_id(0)
    rows = rows_ref[...]
    rowid = pid * ROWS + lax.broadcasted_iota(jnp.int32, (ROWS, 1), 0)
    self_v = jnp.sum(rows * rows, axis=1, keepdims=True)
    lidx = lax.broadcasted_iota(jnp.int32, (ROWS, CHUNK), 1)
    dr = lax.broadcasted_iota(jnp.int32, (CHUNK, 3), 0)
    dc = lax.broadcasted_iota(jnp.int32, (CHUNK, 3), 1)
    digits = jnp.where(
        dc == 0, dr % 16, jnp.where(dc == 1, (dr // 16) % 16, dr // 256)
    ).astype(jnp.float32)
    KO = K - 1
    vs, ix = [], []
    for c in range(N // CHUNK):
        base = c * CHUNK
        cols = colsT_ref[:, pl.ds(base, CHUNK)]
        sim = jnp.dot(rows, cols, preferred_element_type=jnp.float32)
        vals = jnp.where(lidx == rowid - base, NEG, sim)
        s_prev = jnp.dot(vals, digits, preferred_element_type=jnp.float32)
        for r in range(KO):
            m = jnp.max(vals, axis=1, keepdims=True)
            eq = vals == m
            vals = jnp.where(eq, NEG, vals)
            s_new = jnp.dot(vals, digits, preferred_element_type=jnp.float32)
            d = (s_prev - s_new) / (m - NEG)
            di = jnp.floor(d + 0.5).astype(jnp.int32)
            mi = di[:, 2:3] * 256 + di[:, 1:2] * 16 + di[:, 0:1]
            s_prev = s_new
            vs.append(m)
            ix.append(mi + base)
    cand_v = jnp.concatenate(vs, axis=1)
    cand_i = jnp.concatenate(ix, axis=1)
    vs, ix = [], []
    for _ in range(KO):
        m = jnp.max(cand_v, axis=1, keepdims=True)
        eq = cand_v == m
        mi = jnp.min(jnp.where(eq, cand_i, N), axis=1, keepdims=True)
        cand_v = jnp.where(eq & (cand_i == mi), NEG, cand_v)
        vs.append(m)
        ix.append(mi)
    top_v = jnp.concatenate([self_v] + vs, axis=1)
    top_i = jnp.concatenate([rowid] + ix, axis=1)
    top_i = jnp.clip(top_i, 0, N - 1)
    w = top_v * top_v * jnp.float32(0.2)
    w_ref[...] = jnp.concatenate([w, jnp.zeros((ROWS, 8 - K), jnp.float32)], axis=1)
    idx_ref[...] = top_i


def _topk(norm_feat, norm_feat_T):
    return pl.pallas_call(
        _topk_body,
        grid=(N // ROWS,),
        in_specs=[
            pl.BlockSpec((ROWS, D), lambda i: (i, 0)),
            pl.BlockSpec((D, N), lambda i: (0, 0)),
        ],
        out_specs=[
            pl.BlockSpec((ROWS, 8), lambda i: (i, 0)),
            pl.BlockSpec((ROWS, K), lambda i: (i, 0)),
        ],
        out_shape=[
            jax.ShapeDtypeStruct((N, 8), jnp.float32),
            jax.ShapeDtypeStruct((N, K), jnp.int32),
        ],
    )(norm_feat, norm_feat_T)


def _combine_body(table_hbm, idx_hbm, w_hbm, out_hbm, idx_v, rows_v, w_v, out_v, sem):
    c = lax.axis_index("c")
    s = lax.axis_index("s")
    wid = s * 2 + c
    pltpu.sync_copy(idx_hbm.at[wid], idx_v)
    pltpu.sync_copy(w_hbm.at[wid], w_v)
    copies = []
    for b in range(NBLK):
        copies.append(
            pltpu.async_copy(
                table_hbm.at[idx_v.at[b]],
                rows_v.at[pl.ds(b * GBLK, GBLK)],
                sem,
            )
        )
    for cp in copies:
        cp.wait()

    def group_body(g, _):
        wvecs = [w_v[pl.ds(g * 128 + k * 16, 16)] for k in range(8)]
        for rr in range(16):
            acc = jnp.zeros((D,), jnp.float32)
            for j in range(K):
                t = g * (16 * K) + rr * K + j
                p = rr * 8 + j
                acc = acc + rows_v[t, :] * wvecs[p // 16][p % 16]
            out_v[g * 16 + rr, :] = acc
        return 0

    lax.fori_loop(0, ROWS_SC // 16, group_body, 0)
    pltpu.sync_copy(out_v, out_hbm.at[pl.ds(wid * ROWS_SC, ROWS_SC)])


@functools.lru_cache(maxsize=1)
def _combine_kernel():
    return pl.kernel(
        _combine_body,
        out_type=jax.ShapeDtypeStruct((N, D), jnp.float32),
        mesh=plsc.VectorSubcoreMesh(core_axis_name="c", subcore_axis_name="s"),
        scratch_types=[
            pltpu.VMEM((NBLK, GBLK), jnp.int32),
            pltpu.VMEM((TERMS, D), jnp.float32),
            pltpu.VMEM((ROWS_SC * 8,), jnp.float32),
            pltpu.VMEM((ROWS_SC, D), jnp.float32),
            pltpu.SemaphoreType.DMA,
        ],
        compiler_params=pltpu.CompilerParams(use_tc_tiling_on_sc=False),
    )


def kernel(all_feat):
    norm_feat = pl.pallas_call(
        _norm_body,
        out_shape=jax.ShapeDtypeStruct((N, D), jnp.float32),
    )(all_feat)
    w, idx = _topk(norm_feat, norm_feat.T)
    idx_r = idx.reshape(NUM_WORKERS, NBLK, GBLK)
    w_r = w.reshape(NUM_WORKERS, ROWS_SC * 8)
    return _combine_kernel()(all_feat, idx_r, w_r)
